# baseline (speedup 1.0000x reference)
import functools

import jax
import jax.numpy as jnp
from jax import lax
from jax.experimental import pallas as pl
from jax.experimental.pallas import tpu as pltpu
from jax.experimental.pallas import tpu_sc as plsc

NUM_FIELDS = 26
FIELD_DIM = 100000
EMBED_DIM = 16
BATCH = 16384

NC = 2
NS = 16
NW = NC * NS
BPW = BATCH // NW
CHUNK = 64
NCHUNKS = BPW // CHUNK
ROWS = CHUNK * NUM_FIELDS
WROWS = BPW * NUM_FIELDS


def _fm_body(idx_hbm, emb_hbm, lin_hbm, out_hbm,
             idx_v, rows_v0, rows_v1, lin_v0, lin_v1, vtmp_v, out_v,
             isem, sem0, sem1, lsem0, lsem1):
    wid = lax.axis_index("s") * NC + lax.axis_index("c")
    base = wid * BPW

    rows_bufs = (rows_v0, rows_v1)
    lin_bufs = (lin_v0, lin_v1)
    sems = (sem0, sem1)
    lsems = (lsem0, lsem1)

    pltpu.async_copy(idx_hbm.at[pl.ds(base * NUM_FIELDS, WROWS)], idx_v,
                     isem).wait()

    def fire(k, buf):
        sl = idx_v.at[pl.ds(k * ROWS, ROWS)]
        e = pltpu.async_copy(emb_hbm.at[sl], rows_bufs[buf], sems[buf])
        l = pltpu.async_copy(lin_hbm.at[sl], lin_bufs[buf], lsems[buf])
        return e, l

    handles = [None, None]
    handles[0] = fire(0, 0)

    lanes = lax.iota(jnp.int32, 16)

    for k in range(NCHUNKS):
        cur = k % 2
        nxt = (k + 1) % 2
        if k + 1 < NCHUNKS:
            handles[nxt] = fire(k + 1, nxt)
        he, hl = handles[cur]
        he.wait()
        hl.wait()

        rows = rows_bufs[cur]
        linv = lin_bufs[cur]

        def sample_body(i, _, rows=rows):
            s = rows[i * NUM_FIELDS]
            q = s * s
            for f in range(1, NUM_FIELDS):
                r = rows[i * NUM_FIELDS + f]
                s = s + r
                q = q + r * r
            vtmp_v[pl.ds(i * EMBED_DIM, EMBED_DIM)] = s * s - q
            return 0

        lax.fori_loop(0, CHUNK, sample_body, 0)

        def group_body(g, _, linv=linv, k=k):
            rowid = g * 16 + lanes
            vbase = rowid * EMBED_DIM
            acc = plsc.load_gather(vtmp_v, [vbase])
            for d in range(1, EMBED_DIM):
                acc = acc + plsc.load_gather(vtmp_v, [vbase + d])
            lin_base = rowid * NUM_FIELDS
            lacc = plsc.load_gather(linv, [lin_base])
            for f in range(1, NUM_FIELDS):
                lacc = lacc + plsc.load_gather(linv, [lin_base + f])
            out_v[pl.ds(k * CHUNK + g * 16, 16)] = 0.5 * acc + lacc
            return 0

        lax.fori_loop(0, CHUNK // 16, group_body, 0)

    pltpu.sync_copy(out_v, out_hbm.at[pl.ds(base, BPW)])


_fm_call = pl.kernel(
    _fm_body,
    out_type=jax.ShapeDtypeStruct((BATCH,), jnp.float32),
    mesh=plsc.VectorSubcoreMesh(core_axis_name="c", subcore_axis_name="s"),
    compiler_params=pltpu.CompilerParams(
        needs_layout_passes=False, use_tc_tiling_on_sc=False),
    scratch_types=[
        pltpu.VMEM((WROWS,), jnp.int32),
        pltpu.VMEM((ROWS, EMBED_DIM), jnp.float32),
        pltpu.VMEM((ROWS, EMBED_DIM), jnp.float32),
        pltpu.VMEM((ROWS,), jnp.float32),
        pltpu.VMEM((ROWS,), jnp.float32),
        pltpu.VMEM((CHUNK * EMBED_DIM,), jnp.float32),
        pltpu.VMEM((BPW,), jnp.float32),
        pltpu.SemaphoreType.DMA,
        pltpu.SemaphoreType.DMA,
        pltpu.SemaphoreType.DMA,
        pltpu.SemaphoreType.DMA,
        pltpu.SemaphoreType.DMA,
    ],
)


@jax.jit
def kernel(x, emb_table, lin_weight, bias):
    offsets = (jnp.arange(NUM_FIELDS, dtype=jnp.int32) * FIELD_DIM)[None, :]
    idx = (x + offsets).reshape(-1)
    out = _fm_call(idx, emb_table, lin_weight.reshape(-1))
    return out + bias[0]
